# baseline (speedup 1.0000x reference)
import functools

import jax
import jax.numpy as jnp
from jax import lax
from jax.experimental import pallas as pl
from jax.experimental.pallas import tpu as pltpu
from jax.experimental.pallas import tpu_sc as plsc

_EPS = 1e-3


def _sc_gather(table, idxs):
    _, emb = table.shape
    batch = idxs.shape[0]
    info = plsc.get_sparse_core_info()
    num_workers = info.num_cores * info.num_subcores
    bpw = batch // num_workers
    mesh = plsc.VectorSubcoreMesh(core_axis_name="c", subcore_axis_name="s")

    @functools.partial(
        pl.kernel,
        mesh=mesh,
        out_type=jax.ShapeDtypeStruct((batch, emb), jnp.float32),
        scratch_types=[
            pltpu.VMEM((bpw,), jnp.int32),
            pltpu.VMEM((bpw, emb), jnp.float32),
            pltpu.SemaphoreType.DMA,
        ],
    )
    def gather_k(table_hbm, idx_hbm, out_hbm, idx_v, rows_v, sem):
        wid = lax.axis_index("s") * info.num_cores + lax.axis_index("c")
        base = wid * bpw
        pltpu.sync_copy(idx_hbm.at[pl.ds(base, bpw)], idx_v)
        pltpu.async_copy(table_hbm.at[idx_v], rows_v, sem).wait()
        pltpu.sync_copy(rows_v, out_hbm.at[pl.ds(base, bpw)])

    return gather_k(table, idxs)


def _fused_body(idx_ref, x_ref, cents_ref, out_ref,
                *, n, nchunks, bn, cutoff, nd, sub, kk):
    kt = kk // 4
    f32 = jnp.float32

    n2s = []
    cents2 = []
    for d in range(nd):
        cd = cents_ref[d]
        n2s.append(jnp.sum(cd * cd, axis=1)[None, :])
        cents2.append(cd + cd)
    ones_s = jnp.ones((sub, 1), f32)

    def resp(base, d, sup):
        z = x_ref[pl.ds(base, bn), d * sub:(d + 1) * sub]
        n1 = lax.dot_general(z * z, ones_s, (((1,), (0,)), ((), ())),
                             precision=lax.Precision.HIGHEST)
        dt2 = lax.dot_general(z, cents2[d], (((1,), (1,)), ((), ())),
                              precision=lax.Precision.DEFAULT)
        return (dt2 - n1) - sup, z

    tot = resp(0, 0, n2s[0])[0]
    for d in range(1, nd):
        tot = tot + resp(0, d, n2s[d])[0]
    c = jnp.sum(tot, axis=0, keepdims=True) / (float(nd) * bn)
    n2cs = [n2s[d] + c for d in range(nd)]

    def p1(i, carry):
        s1, s2, s1h, s2h, cnth = carry
        base = i * bn
        w = (idx_ref[pl.ds(base, bn), :] >= cutoff).astype(f32)
        for d in range(nd):
            rc = resp(base, d, n2cs[d])[0]
            rc2 = rc * rc
            s1 = s1 + jnp.sum(rc, axis=0, keepdims=True)
            s2 = s2 + jnp.sum(rc2, axis=0, keepdims=True)
            s1h = s1h + jnp.sum(rc * w, axis=0, keepdims=True)
            s2h = s2h + jnp.sum(rc2 * w, axis=0, keepdims=True)
        cnth = cnth + jnp.sum(w)
        return (s1, s2, s1h, s2h, cnth)

    zrow = jnp.zeros((1, kk), f32)
    s1, s2, s1h, s2h, cnth = lax.fori_loop(
        0, nchunks, p1, (zrow, zrow, zrow, zrow, f32(0.0)))

    denh = cnth * float(nd)
    dent = (float(n) - cnth) * float(nd)
    mh_c = s1h / denh
    sh = jnp.sqrt(s2h / denh - mh_c * mh_c + _EPS)
    mh = c + mh_c
    invh = 1.0 / sh
    mt_c = (s1 - s1h) / dent
    stt = jnp.sqrt((s2 - s2h) / dent - mt_c * mt_c + _EPS)
    mt = (c + mt_c)[:, :kt]
    invt = (1.0 / stt)[:, :kt]

    chi = [cents_ref[d].astype(jnp.bfloat16) for d in range(nd)]
    clo = [(cents_ref[d] - chi[d].astype(f32)).astype(jnp.bfloat16)
           for d in range(nd)]

    iota = lax.broadcasted_iota(jnp.int32, (bn, kk), 1)

    def amax_first(a, width):
        maxv = jnp.max(a, axis=1, keepdims=True)
        return jnp.min(jnp.where(a == maxv, iota[:, :width], width),
                       axis=1)

    def p2(i, _):
        base = i * bn
        head = idx_ref[pl.ds(base, bn), :] >= cutoff
        for d in range(nd):
            r, z = resp(base, d, n2s[d])
            rh = (r - mh) * invh
            code_h = amax_first(rh, kk)
            rt = (r[:, :kt] - mt) * invt
            code_t = amax_first(rt, kt)
            code = jnp.where(head, code_h[:, None], code_t[:, None])
            oh = (iota == code).astype(jnp.bfloat16)
            od = (lax.dot_general(oh, chi[d], (((1,), (0,)), ((), ())),
                                  preferred_element_type=f32)
                  + lax.dot_general(oh, clo[d], (((1,), (0,)), ((), ())),
                                    preferred_element_type=f32))
            out_ref[pl.ds(base, bn), d * sub:(d + 1) * sub] = (od - z) + z
        return 0

    lax.fori_loop(0, nchunks, p2, 0)


def kernel(table, centroids, indices):
    vocab, emb = table.shape
    nd, kk, sub = centroids.shape
    cutoff = int(vocab * 0.8)
    idxs = indices.reshape(-1)
    n = idxs.shape[0]

    x = _sc_gather(table, idxs)
    idx2 = idxs[:, None]

    bn = 1024
    nchunks = n // bn
    out = pl.pallas_call(
        functools.partial(_fused_body, n=n, nchunks=nchunks, bn=bn,
                          cutoff=cutoff, nd=nd, sub=sub, kk=kk),
        out_shape=jax.ShapeDtypeStruct((n, emb), jnp.float32),
    )(idx2, x, centroids)

    return out.reshape(indices.shape + (emb,))


# speedup vs baseline: 1.1271x; 1.1271x over previous
import functools

import jax
import jax.numpy as jnp
from jax import lax
from jax.experimental import pallas as pl
from jax.experimental.pallas import tpu as pltpu
from jax.experimental.pallas import tpu_sc as plsc

_EPS = 1e-3


def _sc_gather(table, idxs):
    _, emb = table.shape
    batch = idxs.shape[0]
    info = plsc.get_sparse_core_info()
    num_workers = info.num_cores * info.num_subcores
    bpw = batch // num_workers
    mesh = plsc.VectorSubcoreMesh(core_axis_name="c", subcore_axis_name="s")

    @functools.partial(
        pl.kernel,
        mesh=mesh,
        out_type=jax.ShapeDtypeStruct((batch, emb), jnp.float32),
        scratch_types=[
            pltpu.VMEM((bpw,), jnp.int32),
            pltpu.VMEM((bpw, emb), jnp.float32),
            pltpu.SemaphoreType.DMA,
        ],
    )
    def gather_k(table_hbm, idx_hbm, out_hbm, idx_v, rows_v, sem):
        wid = lax.axis_index("s") * info.num_cores + lax.axis_index("c")
        base = wid * bpw
        pltpu.sync_copy(idx_hbm.at[pl.ds(base, bpw)], idx_v)
        pltpu.async_copy(table_hbm.at[idx_v], rows_v, sem).wait()
        pltpu.sync_copy(rows_v, out_hbm.at[pl.ds(base, bpw)])

    return gather_k(table, idxs)


def _fused_body(w_ref, x_ref, cents_ref, out_ref,
                *, n, nchunks, bn, nd, sub, kk):
    kt = kk // 4
    f32 = jnp.float32

    n2s = []
    cents2 = []
    for d in range(nd):
        cd = cents_ref[d]
        n2s.append(jnp.sum(cd * cd, axis=1)[None, :])
        cents2.append(cd + cd)
    ones_s = jnp.ones((sub, 1), f32)

    def resp(base, d, sup):
        z = x_ref[pl.ds(base, bn), d * sub:(d + 1) * sub]
        n1 = lax.dot_general(z * z, ones_s, (((1,), (0,)), ((), ())),
                             precision=lax.Precision.HIGHEST)
        dt2 = lax.dot_general(z, cents2[d], (((1,), (1,)), ((), ())),
                              precision=lax.Precision.DEFAULT)
        return (dt2 - n1) - sup, z

    tot = resp(0, 0, n2s[0])[0]
    for d in range(1, nd):
        tot = tot + resp(0, d, n2s[d])[0]
    c = jnp.sum(tot, axis=0, keepdims=True) / (float(nd) * bn)
    n2cs = [n2s[d] + c for d in range(nd)]

    def p1(i, carry):
        s1, s2, s1h, s2h, cnth = carry
        base = i * bn
        w = w_ref[pl.ds(base, bn), :].astype(f32)
        for d in range(nd):
            rc = resp(base, d, n2cs[d])[0]
            rc2 = rc * rc
            s1 = s1 + jnp.sum(rc, axis=0, keepdims=True)
            s2 = s2 + jnp.sum(rc2, axis=0, keepdims=True)
            s1h = s1h + jnp.sum(rc * w, axis=0, keepdims=True)
            s2h = s2h + jnp.sum(rc2 * w, axis=0, keepdims=True)
        cnth = cnth + jnp.sum(w)
        return (s1, s2, s1h, s2h, cnth)

    zrow = jnp.zeros((1, kk), f32)
    s1, s2, s1h, s2h, cnth = lax.fori_loop(
        0, nchunks, p1, (zrow, zrow, zrow, zrow, f32(0.0)))

    denh = cnth * float(nd)
    dent = (float(n) - cnth) * float(nd)
    mh_c = s1h / denh
    sh = jnp.sqrt(s2h / denh - mh_c * mh_c + _EPS)
    mh = c + mh_c
    invh = 1.0 / sh
    mt_c = (s1 - s1h) / dent
    stt = jnp.sqrt((s2 - s2h) / dent - mt_c * mt_c + _EPS)
    mt = (c + mt_c)[:, :kt]
    invt = (1.0 / stt)[:, :kt]

    chi = [cents_ref[d].astype(jnp.bfloat16) for d in range(nd)]
    clo = [(cents_ref[d] - chi[d].astype(f32)).astype(jnp.bfloat16)
           for d in range(nd)]

    zpad = jnp.zeros((bn, kk - kt), jnp.bfloat16)

    def p2(i, _):
        base = i * bn
        head = w_ref[pl.ds(base, bn), :] > 0
        for d in range(nd):
            r, z = resp(base, d, n2s[d])
            rh = (r - mh) * invh
            ohh = (rh == jnp.max(rh, axis=1, keepdims=True)
                   ).astype(jnp.bfloat16)
            rt = (r[:, :kt] - mt) * invt
            oht = (rt == jnp.max(rt, axis=1, keepdims=True)
                   ).astype(jnp.bfloat16)
            ohtp = jnp.concatenate([oht, zpad], axis=1)
            oh = jnp.where(head, ohh, ohtp)
            od = (lax.dot_general(oh, chi[d], (((1,), (0,)), ((), ())),
                                  preferred_element_type=f32)
                  + lax.dot_general(oh, clo[d], (((1,), (0,)), ((), ())),
                                    preferred_element_type=f32))
            out_ref[pl.ds(base, bn), d * sub:(d + 1) * sub] = (od - z) + z
        return 0

    lax.fori_loop(0, nchunks, p2, 0)


def kernel(table, centroids, indices):
    vocab, emb = table.shape
    nd, kk, sub = centroids.shape
    cutoff = int(vocab * 0.8)
    idxs = indices.reshape(-1)
    n = idxs.shape[0]

    x = _sc_gather(table, idxs)
    wcol = (idxs >= cutoff).astype(jnp.bfloat16)[:, None]

    bn = 2048
    nchunks = n // bn
    out = pl.pallas_call(
        functools.partial(_fused_body, n=n, nchunks=nchunks, bn=bn,
                          nd=nd, sub=sub, kk=kk),
        out_shape=jax.ShapeDtypeStruct((n, emb), jnp.float32),
    )(wcol, x, centroids)

    return out.reshape(indices.shape + (emb,))


# speedup vs baseline: 1.1988x; 1.0637x over previous
import functools

import jax
import jax.numpy as jnp
from jax import lax
from jax.experimental import pallas as pl
from jax.experimental.pallas import tpu as pltpu
from jax.experimental.pallas import tpu_sc as plsc

_EPS = 1e-3


def _sc_gather(table, idxs):
    _, emb = table.shape
    batch = idxs.shape[0]
    info = plsc.get_sparse_core_info()
    num_workers = info.num_cores * info.num_subcores
    bpw = batch // num_workers
    mesh = plsc.VectorSubcoreMesh(core_axis_name="c", subcore_axis_name="s")

    @functools.partial(
        pl.kernel,
        mesh=mesh,
        out_type=jax.ShapeDtypeStruct((batch, emb), jnp.float32),
        scratch_types=[
            pltpu.VMEM((bpw,), jnp.int32),
            pltpu.VMEM((bpw, emb), jnp.float32),
            pltpu.SemaphoreType.DMA,
        ],
    )
    def gather_k(table_hbm, idx_hbm, out_hbm, idx_v, rows_v, sem):
        wid = lax.axis_index("s") * info.num_cores + lax.axis_index("c")
        base = wid * bpw
        pltpu.sync_copy(idx_hbm.at[pl.ds(base, bpw)], idx_v)
        pltpu.async_copy(table_hbm.at[idx_v], rows_v, sem).wait()
        pltpu.sync_copy(rows_v, out_hbm.at[pl.ds(base, bpw)])

    return gather_k(table, idxs)


def _fused_body(w_ref, x_ref, cents_ref, ctr_ref, out_ref,
                *, n, nchunks, bn, nd, sub, kk):
    kt = kk // 4
    f32 = jnp.float32
    emb = nd * sub

    n2s = []
    cents2 = []
    for d in range(nd):
        cd = cents_ref[d]
        n2s.append(jnp.sum(cd * cd, axis=1)[None, :])
        cents2.append(cd + cd)
    ones_s = jnp.ones((sub, 1), f32)
    onescol = jnp.ones((bn, 1), f32)
    bsel = (lax.broadcasted_iota(jnp.int32, (emb, nd), 0) // sub
            == lax.broadcasted_iota(jnp.int32, (emb, nd), 1)).astype(f32)

    hi = lax.Precision.HIGHEST
    c00 = (((0,), (0,)), ((), ()))
    c10 = (((1,), (0,)), ((), ()))
    c11 = (((1,), (1,)), ((), ()))

    def resp_dt(base, d):
        z = x_ref[pl.ds(base, bn), d * sub:(d + 1) * sub]
        n1 = lax.dot_general(z * z, ones_s, c10, precision=hi)
        dt2 = lax.dot_general(z, cents2[d], c11,
                              precision=lax.Precision.DEFAULT)
        return dt2 - n1, z

    tot = resp_dt(0, 0)[0] - n2s[0]
    for d in range(1, nd):
        tot = tot + (resp_dt(0, d)[0] - n2s[d])
    c = jnp.sum(tot, axis=0, keepdims=True) / (float(nd) * bn)
    qs = [n2s[d] + c for d in range(nd)]

    def p1(i, carry):
        (gh, gt, znh, znt, zsh, zst, a1h, a1t, q1h, q1t, cnth) = carry
        base = i * bn
        w = w_ref[pl.ds(base, bn), :].astype(f32)
        xb = x_ref[pl.ds(base, bn), :]
        xw = xb * w
        x2 = xb * xb
        n14 = lax.dot_general(x2, bsel, c10, precision=hi)
        n14w = n14 * w
        nsq = n14 * n14
        gh = gh + lax.dot_general(xw, xb, c00, precision=hi)
        gt = gt + lax.dot_general(xb, xb, c00, precision=hi)
        znh = znh + lax.dot_general(n14w, xb, c00, precision=hi)
        znt = znt + lax.dot_general(n14, xb, c00, precision=hi)
        zsh = zsh + lax.dot_general(w, xb, c00, precision=hi)
        zst = zst + lax.dot_general(onescol, xb, c00, precision=hi)
        a1h = a1h + lax.dot_general(w, n14, c00, precision=hi)
        a1t = a1t + lax.dot_general(onescol, n14, c00, precision=hi)
        q1h = q1h + lax.dot_general(w, nsq, c00, precision=hi)
        q1t = q1t + lax.dot_general(onescol, nsq, c00, precision=hi)
        cnth = cnth + jnp.sum(w)
        return (gh, gt, znh, znt, zsh, zst, a1h, a1t, q1h, q1t, cnth)

    zg = jnp.zeros((emb, emb), f32)
    zn0 = jnp.zeros((nd, emb), f32)
    zr0 = jnp.zeros((1, emb), f32)
    z40 = jnp.zeros((1, nd), f32)
    (gh, gt, znh, znt, zsh, zst, a1h, a1t, q1h, q1t, cnth) = lax.fori_loop(
        0, nchunks, p1,
        (zg, zg, zn0, zn0, zr0, zr0, z40, z40, z40, z40, f32(0.0)))

    def contrib(gx, znx, zsx, a1x, q1x, cnt, d):
        cdt = ctr_ref[d]
        sl = slice(d * sub, (d + 1) * sub)
        q = qs[d]
        zs = zsx[:, sl]
        zn = znx[d:d + 1, sl]
        a1 = a1x[:, d:d + 1]
        q1 = q1x[:, d:d + 1]
        czs = lax.dot_general(zs, cdt, c10, precision=hi)
        czn = lax.dot_general(zn, cdt, c10, precision=hi)
        t1 = lax.dot_general(gx[sl, sl], cdt, c10, precision=hi)
        quad = lax.dot_general(jnp.ones((1, sub), f32), t1 * cdt, c10,
                               precision=hi)
        s1 = 2.0 * czs - a1 - cnt * q
        s2 = (4.0 * quad - 4.0 * (czn + q * czs)
              + q1 + 2.0 * q * a1 + cnt * q * q)
        return s1, s2

    denh = cnth * float(nd)
    dent = (float(n) - cnth) * float(nd)
    s1hp = s2hp = s1tp = s2tp = None
    for d in range(nd):
        a, b = contrib(gh, znh, zsh, a1h, q1h, cnth, d)
        s1hp = a if s1hp is None else s1hp + a
        s2hp = b if s2hp is None else s2hp + b
        a, b = contrib(gt - gh, znt - znh, zst - zsh, a1t - a1h,
                       q1t - q1h, float(n) - cnth, d)
        s1tp = a if s1tp is None else s1tp + a
        s2tp = b if s2tp is None else s2tp + b
    mh_c = s1hp / denh
    sh = jnp.sqrt(s2hp / denh - mh_c * mh_c + _EPS)
    invh_row = 1.0 / sh
    mt_c = s1tp / dent
    st_ = jnp.sqrt(s2tp / dent - mt_c * mt_c + _EPS)
    invt_row = (1.0 / st_)[:, :kt]
    mh = [n2s[d] + (c + mh_c) for d in range(nd)]
    mt = [n2s[d][:, :kt] + (c + mt_c)[:, :kt] for d in range(nd)]
    invh = [invh_row] * nd
    invt = [invt_row] * nd

    chi = [cents_ref[d].astype(jnp.bfloat16) for d in range(nd)]
    clo = [(cents_ref[d] - chi[d].astype(f32)).astype(jnp.bfloat16)
           for d in range(nd)]
    zpad = jnp.zeros((bn, kk - kt), jnp.bfloat16)

    def p2(i, _):
        base = i * bn
        head = w_ref[pl.ds(base, bn), :] > 0
        for d in range(nd):
            t, z = resp_dt(base, d)
            rh = (t - mh[d]) * invh[d]
            ohh = (rh == jnp.max(rh, axis=1, keepdims=True)
                   ).astype(jnp.bfloat16)
            rt = (t[:, :kt] - mt[d]) * invt[d]
            oht = (rt == jnp.max(rt, axis=1, keepdims=True)
                   ).astype(jnp.bfloat16)
            oh = jnp.where(head, ohh, jnp.concatenate([oht, zpad], axis=1))
            od = (lax.dot_general(oh, chi[d], c10, preferred_element_type=f32)
                  + lax.dot_general(oh, clo[d], c10,
                                    preferred_element_type=f32))
            out_ref[pl.ds(base, bn), d * sub:(d + 1) * sub] = (od - z) + z
        return 0

    lax.fori_loop(0, nchunks, p2, 0)


def kernel(table, centroids, indices):
    vocab, emb = table.shape
    nd, kk, sub = centroids.shape
    cutoff = int(vocab * 0.8)
    idxs = indices.reshape(-1)
    n = idxs.shape[0]

    x = _sc_gather(table, idxs)
    wcol = (idxs >= cutoff).astype(jnp.bfloat16)[:, None]
    centsT = centroids.transpose(0, 2, 1)

    bn = 2048
    nchunks = n // bn
    out = pl.pallas_call(
        functools.partial(_fused_body, n=n, nchunks=nchunks, bn=bn,
                          nd=nd, sub=sub, kk=kk),
        out_shape=jax.ShapeDtypeStruct((n, emb), jnp.float32),
    )(wcol, x, centroids, centsT)

    return out.reshape(indices.shape + (emb,))
